# baseline (speedup 1.0000x reference)
import functools

import jax
import jax.numpy as jnp
from jax.experimental import pallas as pl
from jax.experimental.pallas import tpu as pltpu

HIDDEN = 1024
N_EXPERTS = 8
INTER = 1408
TOP_K = 2
T = 4096

TILE_T = 512


def _routing_kernel(logits_ref, w_ref):
    logits = logits_ref[...]
    m = jnp.max(logits, axis=-1, keepdims=True)
    p = jnp.exp(logits - m)
    p = p / jnp.sum(p, axis=-1, keepdims=True)
    iota = jax.lax.broadcasted_iota(jnp.int32, p.shape, 1)
    p1 = jnp.max(p, axis=-1, keepdims=True)
    is1 = p == p1
    first1 = iota == jnp.min(jnp.where(is1, iota, N_EXPERTS), axis=-1, keepdims=True)
    p_wo = jnp.where(first1, -jnp.inf, p)
    p2 = jnp.max(p_wo, axis=-1, keepdims=True)
    is2 = p_wo == p2
    first2 = iota == jnp.min(jnp.where(is2, iota, N_EXPERTS), axis=-1, keepdims=True)
    denom = p1 + p2
    w_ref[...] = jnp.where(first1, p1 / denom, 0.0) + jnp.where(first2, p2 / denom, 0.0)


def _moe_dense_kernel(x_ref, wgt_ref, w1_ref, w2_ref, out_ref):
    e = pl.program_id(1)
    xb = x_ref[...].astype(jnp.bfloat16)
    h = jnp.dot(xb, w1_ref[0].T, preferred_element_type=jnp.float32)
    h = h * jax.nn.sigmoid(h)
    y = jnp.dot(h.astype(jnp.bfloat16), w2_ref[0].T, preferred_element_type=jnp.float32)
    wgt = wgt_ref[...]
    eiota = jax.lax.broadcasted_iota(jnp.int32, wgt.shape, 1)
    wcol = jnp.sum(jnp.where(eiota == e, wgt, 0.0), axis=1)
    y = y * wcol[:, None]

    @pl.when(e == 0)
    def _():
        out_ref[...] = y

    @pl.when(e > 0)
    def _():
        out_ref[...] += y


@jax.jit
def kernel(x, router_logits, w1, w2):
    wgt = pl.pallas_call(
        _routing_kernel,
        out_shape=jax.ShapeDtypeStruct((T, N_EXPERTS), jnp.float32),
    )(router_logits)

    out = pl.pallas_call(
        _moe_dense_kernel,
        grid=(T // TILE_T, N_EXPERTS),
        in_specs=[
            pl.BlockSpec((TILE_T, HIDDEN), lambda i, e: (i, 0)),
            pl.BlockSpec((TILE_T, N_EXPERTS), lambda i, e: (i, 0)),
            pl.BlockSpec((1, INTER, HIDDEN), lambda i, e: (e, 0, 0)),
            pl.BlockSpec((1, HIDDEN, INTER), lambda i, e: (e, 0, 0)),
        ],
        out_specs=pl.BlockSpec((TILE_T, HIDDEN), lambda i, e: (i, 0)),
        out_shape=jax.ShapeDtypeStruct((T, HIDDEN), jnp.float32),
    )(x, wgt, w1.astype(jnp.bfloat16), w2.astype(jnp.bfloat16))
    return out


# speedup vs baseline: 1.4316x; 1.4316x over previous
import functools

import jax
import jax.numpy as jnp
from jax import lax
from jax.experimental import pallas as pl
from jax.experimental.pallas import tpu as pltpu
from jax.experimental.pallas import tpu_sc as plsc

HIDDEN = 1024
N_EXPERTS = 8
INTER = 1408
T = 4096

BT = 256
G = 2 * T // BT + N_EXPERTS
PAD = G * BT
NW = 32
A = 2 * T
APW = A // NW
SUB = 64
TPW = T // NW
CSUB = 32
CHUNK = 512


def _route_kernel(logits_ref, pos_ref, wrep_ref, meta_ref):
    logits = logits_ref[...]
    m = jnp.max(logits, axis=-1, keepdims=True)
    p = jnp.exp(logits - m)
    p = p / jnp.sum(p, axis=-1, keepdims=True)
    iota = lax.broadcasted_iota(jnp.int32, p.shape, 1)
    p1 = jnp.max(p, axis=-1, keepdims=True)
    is1 = p == p1
    first1 = iota == jnp.min(jnp.where(is1, iota, N_EXPERTS), axis=-1, keepdims=True)
    p_wo = jnp.where(first1, -jnp.inf, p)
    p2 = jnp.max(p_wo, axis=-1, keepdims=True)
    is2 = p_wo == p2
    first2 = iota == jnp.min(jnp.where(is2, iota, N_EXPERTS), axis=-1, keepdims=True)
    denom = p1 + p2
    S = first1.astype(jnp.float32) + first2.astype(jnp.float32)

    counts = jnp.sum(S, axis=0, keepdims=True)
    nb = jnp.floor((counts + (BT - 1.0)) * (1.0 / BT))
    ii = lax.broadcasted_iota(jnp.int32, (N_EXPERTS, N_EXPERTS), 0)
    jj = lax.broadcasted_iota(jnp.int32, (N_EXPERTS, N_EXPERTS), 1)
    bs = jnp.dot(nb, (ii < jj).astype(jnp.float32),
                 preferred_element_type=jnp.float32)
    base = bs * float(BT)

    biota = lax.broadcasted_iota(jnp.int32, (N_EXPERTS, 128), 1).astype(jnp.float32)
    be = jnp.sum((jnp.broadcast_to(bs.T, (N_EXPERTS, 128)) <= biota)
                 .astype(jnp.float32), axis=0) - 1.0
    meta_ref[...] = be[None, :].astype(jnp.int32)

    wrep_ref[0:T, :] = jnp.broadcast_to(p1 / denom, (T, 128))
    wrep_ref[T:A, :] = jnp.broadcast_to(p2 / denom, (T, 128))

    tri = (lax.broadcasted_iota(jnp.int32, (CHUNK, CHUNK), 0)
           >= lax.broadcasted_iota(jnp.int32, (CHUNK, CHUNK), 1)).astype(jnp.float32)
    running = jnp.zeros((1, N_EXPERTS), jnp.float32)
    p0_chunks, p1_chunks = [], []
    for c in range(T // CHUNK):
        Sc = S[c * CHUNK:(c + 1) * CHUNK, :]
        rank = jnp.dot(tri, Sc, preferred_element_type=jnp.float32) + running
        running = running + jnp.sum(Sc, axis=0, keepdims=True)
        slot = jnp.broadcast_to(base, rank.shape) + rank - 1.0
        f1c = first1[c * CHUNK:(c + 1) * CHUNK, :]
        f2c = first2[c * CHUNK:(c + 1) * CHUNK, :]
        p0_chunks.append(jnp.sum(jnp.where(f1c, slot, 0.0), axis=1))
        p1_chunks.append(jnp.sum(jnp.where(f2c, slot, 0.0), axis=1))
    pos0 = jnp.concatenate(p0_chunks)
    pos1 = jnp.concatenate(p1_chunks)
    pos_ref[...] = jnp.stack([pos0, pos1]).astype(jnp.int32)


@functools.lru_cache(maxsize=1)
def _sc_kernels():
    mesh = plsc.VectorSubcoreMesh(core_axis_name="c", subcore_axis_name="s")

    @functools.partial(
        pl.kernel,
        out_type=[jax.ShapeDtypeStruct((PAD, HIDDEN), jnp.float32),
                  jax.ShapeDtypeStruct((PAD, 128), jnp.float32)],
        mesh=mesh,
        scratch_types=[
            pltpu.VMEM((SUB,), jnp.int32),
            pltpu.VMEM((SUB, HIDDEN), jnp.float32),
            pltpu.VMEM((SUB, 128), jnp.float32),
            pltpu.SemaphoreType.DMA,
            pltpu.SemaphoreType.DMA,
        ],
    )
    def dispatch(x_hbm, pos_hbm, wrep_hbm, xg_hbm, wg_hbm,
                 idx_v, rows_v, wv_v, sem0, sem1):
        wid = lax.axis_index("s") * 2 + lax.axis_index("c")
        base_a = wid * APW
        for j in range(APW // SUB):
            a0 = base_a + j * SUB
            t0 = lax.rem(a0, T)
            pltpu.sync_copy(pos_hbm.at[pl.ds(a0, SUB)], idx_v)
            pltpu.sync_copy(x_hbm.at[pl.ds(t0, SUB)], rows_v)
            pltpu.sync_copy(wrep_hbm.at[pl.ds(a0, SUB)], wv_v)
            c0 = pltpu.async_copy(rows_v, xg_hbm.at[idx_v], sem0)
            c1 = pltpu.async_copy(wv_v, wg_hbm.at[idx_v], sem1)
            c0.wait()
            c1.wait()

    @functools.partial(
        pl.kernel,
        out_type=jax.ShapeDtypeStruct((T, HIDDEN), jnp.float32),
        mesh=mesh,
        scratch_types=[
            pltpu.VMEM((CSUB,), jnp.int32),
            pltpu.VMEM((CSUB,), jnp.int32),
            pltpu.VMEM((CSUB, HIDDEN), jnp.float32),
            pltpu.VMEM((CSUB, HIDDEN), jnp.float32),
            pltpu.SemaphoreType.DMA,
            pltpu.SemaphoreType.DMA,
        ],
    )
    def combine(yg_hbm, pos_hbm, out_hbm, idx0_v, idx1_v, r0_v, r1_v, sem0, sem1):
        wid = lax.axis_index("s") * 2 + lax.axis_index("c")
        for j in range(TPW // CSUB):
            t0 = wid * TPW + j * CSUB
            pltpu.sync_copy(pos_hbm.at[pl.ds(t0, CSUB)], idx0_v)
            pltpu.sync_copy(pos_hbm.at[pl.ds(T + t0, CSUB)], idx1_v)
            c0 = pltpu.async_copy(yg_hbm.at[idx0_v], r0_v, sem0)
            c1 = pltpu.async_copy(yg_hbm.at[idx1_v], r1_v, sem1)
            c0.wait()
            c1.wait()

            def row_body(r, _):
                for q in range(4):
                    for u in range(16):
                        off = q * 256 + u * 16
                        r0_v[r, pl.ds(off, 16)] = (r0_v[r, pl.ds(off, 16)]
                                                   + r1_v[r, pl.ds(off, 16)])
                return 0

            lax.fori_loop(0, CSUB, row_body, 0)
            pltpu.sync_copy(r0_v, out_hbm.at[pl.ds(t0, CSUB)])

    return dispatch, combine


def _gmm_kernel(be_ref, xg_ref, w1_ref, w2_ref, wg_ref, yg_ref):
    del be_ref
    h = jnp.dot(xg_ref[...], w1_ref[0].T, preferred_element_type=jnp.float32)
    h = h * jax.nn.sigmoid(h)
    y = jnp.dot(h, w2_ref[0].T, preferred_element_type=jnp.float32)
    yg_ref[...] = y * wg_ref[:, 0:1]


@jax.jit
def kernel(x, router_logits, w1, w2):
    pos, wrep, meta = pl.pallas_call(
        _route_kernel,
        out_shape=[
            jax.ShapeDtypeStruct((2, T), jnp.int32),
            jax.ShapeDtypeStruct((A, 128), jnp.float32),
            jax.ShapeDtypeStruct((1, 128), jnp.int32),
        ],
    )(router_logits)
    be = meta[0, :G]
    pos_flat = pos.reshape(A)

    dispatch, combine = _sc_kernels()
    xg, wg = dispatch(x, pos_flat, wrep)

    yg = pl.pallas_call(
        _gmm_kernel,
        grid_spec=pltpu.PrefetchScalarGridSpec(
            num_scalar_prefetch=1,
            grid=(G,),
            in_specs=[
                pl.BlockSpec((BT, HIDDEN), lambda g, be_r: (g, 0)),
                pl.BlockSpec((1, INTER, HIDDEN), lambda g, be_r: (be_r[g], 0, 0)),
                pl.BlockSpec((1, HIDDEN, INTER), lambda g, be_r: (be_r[g], 0, 0)),
                pl.BlockSpec((BT, 128), lambda g, be_r: (g, 0)),
            ],
            out_specs=pl.BlockSpec((BT, HIDDEN), lambda g, be_r: (g, 0)),
        ),
        out_shape=jax.ShapeDtypeStruct((PAD, HIDDEN), jnp.float32),
    )(be, xg, w1, w2, wg)

    return combine(yg, pos_flat)
